# baseline (speedup 1.0000x reference)
import functools

import jax
import jax.numpy as jnp
from jax import lax
from jax.experimental import pallas as pl
from jax.experimental.pallas import tpu as pltpu
from jax.experimental.pallas import tpu_sc as plsc

N = 10000
E = 320000
D_IN = 128
HID = 64
C = 40
CP = 48
NITER = 10
EXPL = 0.1

NC = 2
NS = 16
NW = NC * NS
K = 128
CHUNKS = 80
NBUF = 10
EPAD = NW * CHUNKS * K
NOUT = 10112
RPT = NOUT // NS

BLK = 1000
GRID = N // BLK


_sc_mesh = plsc.VectorSubcoreMesh(
    core_axis_name="c", subcore_axis_name="s", num_cores=NC, num_subcores=NS)


@functools.partial(
    pl.kernel,
    out_type=jax.ShapeDtypeStruct((NC, NOUT, CP), jnp.float32),
    mesh=_sc_mesh,
    scratch_types=[
        pltpu.VMEM((CHUNKS, K), jnp.int32),
        pltpu.VMEM((CHUNKS, K), jnp.int32),
        pltpu.VMEM((NBUF, K, CP), jnp.float32),
        pltpu.VMEM((K, CP), jnp.float32),
        pltpu.VMEM_SHARED((NOUT, CP), jnp.float32),
        pltpu.SemaphoreType.DMA,
    ] + [pltpu.SemaphoreType.DMA] * NBUF,
    compiler_params=pltpu.CompilerParams(use_tc_tiling_on_sc=False),
)
def _sc_scatter(p_hbm, src_hbm, dst_hbm, out_hbm, sall, dall, rows, zbuf, acc,
                gsem, *ssems):
    c = lax.axis_index("c")
    s = lax.axis_index("s")
    wid = s * NC + c

    zero16 = jnp.zeros((16,), jnp.float32)

    def _zrow(i, carry):
        zbuf[i, pl.ds(0, 16)] = zero16
        zbuf[i, pl.ds(16, 16)] = zero16
        zbuf[i, pl.ds(32, 16)] = zero16
        return carry

    lax.fori_loop(0, K, _zrow, 0)
    zb = s * RPT
    for q in range(RPT // K):
        pltpu.sync_copy(zbuf, acc.at[pl.ds(zb + q * K, K)])
    _rem = RPT - (RPT // K) * K
    pltpu.sync_copy(zbuf.at[pl.ds(0, _rem)], acc.at[pl.ds(zb + (RPT // K) * K, _rem)])
    plsc.subcore_barrier()

    pltpu.sync_copy(src_hbm.at[wid], sall)
    pltpu.sync_copy(dst_hbm.at[wid], dall)

    def _wait_gather(buf):
        pltpu.make_async_copy(p_hbm.at[pl.ds(0, K)], buf, gsem).wait()

    def _wait_scatter(buf, sem):
        pltpu.make_async_copy(buf, acc.at[pl.ds(0, K)], sem).wait()

    for b in range(NBUF):
        pltpu.async_copy(p_hbm.at[sall.at[b]], rows.at[b], gsem)

    def _grp(gi, carry):
        for b in range(NBUF):
            j = gi * NBUF + b
            buf = rows.at[b]
            _wait_gather(buf)
            pltpu.async_copy(buf, acc.at[dall.at[j]], ssems[b], add=True)

            @pl.when(j + NBUF < CHUNKS)
            def _():
                _wait_scatter(buf, ssems[b])
                pltpu.async_copy(p_hbm.at[sall.at[j + NBUF]], buf, gsem)

        return carry

    lax.fori_loop(0, CHUNKS // NBUF, _grp, 0)
    for b in range(NBUF):
        _wait_scatter(rows.at[b], ssems[b])
    plsc.subcore_barrier()

    pltpu.sync_copy(acc.at[pl.ds(s * RPT, RPT)], out_hbm.at[c, pl.ds(s * RPT, RPT)])


@functools.partial(
    pl.kernel,
    out_type=jax.ShapeDtypeStruct((NC, NOUT, CP), jnp.float32),
    mesh=_sc_mesh,
    scratch_types=[
        pltpu.VMEM((CHUNKS, K), jnp.int32),
        pltpu.VMEM((K, CP), jnp.float32),
        pltpu.VMEM((K, CP), jnp.float32),
        pltpu.VMEM_SHARED((NOUT, CP), jnp.float32),
    ] + [pltpu.SemaphoreType.DMA] * NBUF,
    compiler_params=pltpu.CompilerParams(use_tc_tiling_on_sc=False),
)
def _sc_degree(dst_hbm, out_hbm, dall, obuf, zbuf, acc, *ssems):
    c = lax.axis_index("c")
    s = lax.axis_index("s")
    wid = s * NC + c

    zero16 = jnp.zeros((16,), jnp.float32)
    one16 = jnp.ones((16,), jnp.float32)

    def _zrow(i, carry):
        for q in range(CP // 16):
            zbuf[i, pl.ds(16 * q, 16)] = zero16
            obuf[i, pl.ds(16 * q, 16)] = one16
        return carry

    lax.fori_loop(0, K, _zrow, 0)
    zb = s * RPT
    for q in range(RPT // K):
        pltpu.sync_copy(zbuf, acc.at[pl.ds(zb + q * K, K)])
    _rem = RPT - (RPT // K) * K
    pltpu.sync_copy(zbuf.at[pl.ds(0, _rem)], acc.at[pl.ds(zb + (RPT // K) * K, _rem)])
    plsc.subcore_barrier()

    pltpu.sync_copy(dst_hbm.at[wid], dall)

    def _wait_scatter(sem):
        pltpu.make_async_copy(obuf, acc.at[pl.ds(0, K)], sem).wait()

    def _grp(gi, carry):
        for b in range(NBUF):
            j = gi * NBUF + b

            @pl.when(j >= NBUF)
            def _():
                _wait_scatter(ssems[b])

            pltpu.async_copy(obuf, acc.at[dall.at[j]], ssems[b], add=True)

        return carry

    lax.fori_loop(0, CHUNKS // NBUF, _grp, 0)
    for b in range(NBUF):
        _wait_scatter(ssems[b])
    plsc.subcore_barrier()

    pltpu.sync_copy(acc.at[pl.ds(s * RPT, RPT)], out_hbm.at[c, pl.ds(s * RPT, RPT)])


def _enc_body(x_ref, w1_ref, b1_ref, w2_ref, b2_ref, z_ref):
    h = jnp.dot(x_ref[...], w1_ref[...], preferred_element_type=jnp.float32)
    h = jnp.maximum(h + b1_ref[...], 0.0)
    z_ref[...] = jnp.dot(h, w2_ref[...], preferred_element_type=jnp.float32) + b2_ref[...]


_encode = pl.pallas_call(
    _enc_body,
    grid=(GRID,),
    in_specs=[
        pl.BlockSpec((BLK, D_IN), lambda i: (i, 0)),
        pl.BlockSpec((D_IN, HID), lambda i: (0, 0)),
        pl.BlockSpec((1, HID), lambda i: (0, 0)),
        pl.BlockSpec((HID, CP), lambda i: (0, 0)),
        pl.BlockSpec((1, CP), lambda i: (0, 0)),
    ],
    out_specs=pl.BlockSpec((BLK, CP), lambda i: (i, 0)),
    out_shape=jax.ShapeDtypeStruct((N, CP), jnp.float32),
)


def _prep_body(z_ref, o_ref, dinvb_ref, p0_ref):
    deg = o_ref[0][:, 0:1] + o_ref[1][:, 0:1] + 1.0
    dinv = jnp.where(deg > 0, 1.0 / jnp.sqrt(deg), 0.0)
    dinvb = jnp.broadcast_to(dinv, (BLK, CP))
    dinvb_ref[...] = dinvb
    p0_ref[...] = z_ref[...] * dinvb


_prep = pl.pallas_call(
    _prep_body,
    grid=(GRID,),
    in_specs=[
        pl.BlockSpec((BLK, CP), lambda i: (i, 0)),
        pl.BlockSpec((NC, BLK, CP), lambda i: (0, i, 0)),
    ],
    out_specs=[
        pl.BlockSpec((BLK, CP), lambda i: (i, 0)),
        pl.BlockSpec((BLK, CP), lambda i: (i, 0)),
    ],
    out_shape=[
        jax.ShapeDtypeStruct((N, CP), jnp.float32),
        jax.ShapeDtypeStruct((N, CP), jnp.float32),
    ],
)


def _head_body(final, t, o_ref, p_ref, prop_ref, dinvb_ref, act_ref, steps_ref,
               hlp_ref, hv_ref, hent_ref, noise_ref, u_ref,
               pw1_ref, pb1_ref, pw2_ref, pb2_ref, pw3_ref, pb3_ref,
               vw1_ref, vb1_ref, vw2_ref, vb2_ref, vw3_ref, vb3_ref, *outs):
    dinvb = dinvb_ref[...]
    newp = dinvb * (o_ref[0] + o_ref[1] + p_ref[...])
    a = act_ref[...] > 0.0
    prop_new = jnp.where(a, newp, prop_ref[...])

    ph = jnp.maximum(jnp.dot(prop_new, pw1_ref[...], preferred_element_type=jnp.float32) + pb1_ref[...], 0.0)
    ph = jnp.maximum(jnp.dot(ph, pw2_ref[...], preferred_element_type=jnp.float32) + pb2_ref[...], 0.0)
    logit = jnp.dot(ph, pw3_ref[...], preferred_element_type=jnp.float32) + pb3_ref[...]
    p = jax.nn.sigmoid(logit)

    vh = jnp.maximum(jnp.dot(prop_new, vw1_ref[...], preferred_element_type=jnp.float32) + vb1_ref[...], 0.0)
    vh = jnp.maximum(jnp.dot(vh, vw2_ref[...], preferred_element_type=jnp.float32) + vb2_ref[...], 0.0)
    v = jnp.dot(vh, vw3_ref[...], preferred_element_type=jnp.float32) + vb3_ref[...]

    ent = -(p * jnp.log(p + 1e-10) + (1.0 - p) * jnp.log(1.0 - p + 1e-10))
    np_ = jnp.clip(p + noise_ref[...], 0.01, 0.99)
    halt = a & (u_ref[...] < np_)
    hlp_n = jnp.where(halt, jnp.log(np_ + 1e-10), hlp_ref[...])
    hv_n = jnp.where(halt, v, hv_ref[...])
    hent_n = jnp.where(halt, ent, hent_ref[...])
    a_n = a & jnp.logical_not(halt)

    if not final:
        prop_o, pnew_o, act_o, steps_o, hlp_o, hv_o, hent_o = outs
        prop_o[...] = prop_new
        pnew_o[...] = prop_new * dinvb
        act_o[...] = jnp.where(a_n, 1.0, 0.0)
        steps_o[...] = jnp.where(a_n, float(t + 2), steps_ref[...])
        hlp_o[...] = hlp_n
        hv_o[...] = hv_n
        hent_o[...] = hent_n
    else:
        out_o, steps_o, hlp_o, hv_o, hent_o = outs
        steps_o[...] = jnp.where(a_n, float(NITER), steps_ref[...])
        hlp_o[...] = jnp.where(a_n, jnp.log(np_ + 1e-10), hlp_n)
        hv_o[...] = jnp.where(a_n, v, hv_n)
        hent_o[...] = jnp.where(a_n, ent, hent_n)
        x40 = prop_new[:, :C]
        m = jnp.max(x40, axis=1, keepdims=True)
        sh = x40 - lax.stop_gradient(m)
        out_o[...] = sh - jnp.log(jnp.sum(jnp.exp(sh), axis=1, keepdims=True))


def _mk_head(final, t):
    mat = lambda: pl.BlockSpec((BLK, CP), lambda i: (i, 0))
    col = lambda: pl.BlockSpec((BLK, 1), lambda i: (i, 0))
    full = lambda r, c_: pl.BlockSpec((r, c_), lambda i: (0, 0))
    in_specs = [
        pl.BlockSpec((NC, BLK, CP), lambda i: (0, i, 0)),
        mat(), mat(), mat(),
        col(), col(), col(), col(), col(), col(), col(),
        full(CP, 20), full(1, 20), full(20, 10), full(1, 10), full(10, 1), full(1, 1),
        full(CP, 20), full(1, 20), full(20, 10), full(1, 10), full(10, 1), full(1, 1),
    ]
    if not final:
        out_specs = [mat(), mat(), col(), col(), col(), col(), col()]
        out_shape = [jax.ShapeDtypeStruct((N, CP), jnp.float32)] * 2 + \
                    [jax.ShapeDtypeStruct((N, 1), jnp.float32)] * 5
    else:
        out_specs = [pl.BlockSpec((BLK, C), lambda i: (i, 0)),
                     col(), col(), col(), col()]
        out_shape = [jax.ShapeDtypeStruct((N, C), jnp.float32)] + \
                    [jax.ShapeDtypeStruct((N, 1), jnp.float32)] * 4
    return pl.pallas_call(
        functools.partial(_head_body, final, t),
        grid=(GRID,),
        in_specs=in_specs,
        out_specs=out_specs,
        out_shape=out_shape,
    )


_head_step = [_mk_head(False, t) for t in range(NITER - 1)]
_head_final = _mk_head(True, NITER - 1)


def kernel(x, edge_index, W1, b1, W2, b2, pW1, pb1, pW2, pb2, pW3, pb3,
           vW1, vb1, vW2, vb2, vW3, vb3):
    f32 = jnp.float32
    src = edge_index[0].astype(jnp.int32)
    dst = edge_index[1].astype(jnp.int32)
    src_p = jnp.concatenate([src, jnp.zeros((EPAD - E,), jnp.int32)]).reshape(NW, CHUNKS, K)
    dst_p = jnp.concatenate([dst, jnp.full((EPAD - E,), N, jnp.int32)]).reshape(NW, CHUNKS, K)

    W2p = jnp.pad(W2, ((0, 0), (0, CP - C)))
    b2p = jnp.pad(b2, (0, CP - C)).reshape(1, CP)
    pW1p = jnp.pad(pW1, ((0, CP - C), (0, 0)))
    vW1p = jnp.pad(vW1, ((0, CP - C), (0, 0)))
    wts = (pW1p, pb1.reshape(1, -1), pW2, pb2.reshape(1, -1), pW3, pb3.reshape(1, -1),
           vW1p, vb1.reshape(1, -1), vW2, vb2.reshape(1, -1), vW3, vb3.reshape(1, -1))

    rkey = jax.random.key(42)
    noises = [(jax.random.normal(jax.random.fold_in(rkey, 2 * t), (N,), f32) * EXPL).reshape(N, 1)
              for t in range(NITER)]
    us = [jax.random.uniform(jax.random.fold_in(rkey, 2 * t + 1), (N,), f32).reshape(N, 1)
          for t in range(NITER)]

    z = _encode(x, W1, b1.reshape(1, HID), W2p, b2p)

    odeg = _sc_degree(dst_p)
    dinvb, P = _prep(z, odeg)

    prop = z
    act = jnp.ones((N, 1), f32)
    steps = jnp.ones((N, 1), f32)
    hlp = jnp.zeros((N, 1), f32)
    hv = jnp.zeros((N, 1), f32)
    hent = jnp.zeros((N, 1), f32)
    for t in range(NITER):
        o = _sc_scatter(P, src_p, dst_p)
        args = (o, P, prop, dinvb, act, steps, hlp, hv, hent, noises[t], us[t]) + wts
        if t < NITER - 1:
            prop, P, act, steps, hlp, hv, hent = _head_step[t](*args)
        else:
            out, steps, hlp, hv, hent = _head_final(*args)
    return (out, steps[:, 0], hlp[:, 0], hv[:, 0], hent[:, 0])


# speedup vs baseline: 1.0791x; 1.0791x over previous
import functools

import jax
import jax.numpy as jnp
from jax import lax
from jax.experimental import pallas as pl
from jax.experimental.pallas import tpu as pltpu
from jax.experimental.pallas import tpu_sc as plsc

N = 10000
E = 320000
D_IN = 128
HID = 64
C = 40
CP = 48
NITER = 10
EXPL = 0.1

NC = 2
NS = 16
NW = NC * NS
K = 128
CHUNKS = 80
NBUF = 10
EPAD = NW * CHUNKS * K
NSRC = N + 16
NOUT = 10112
RPT = NOUT // NS

BLK = 1000
GRID = N // BLK


_sc_mesh = plsc.VectorSubcoreMesh(
    core_axis_name="c", subcore_axis_name="s", num_cores=NC, num_subcores=NS)


@functools.partial(
    pl.kernel,
    out_type=jax.ShapeDtypeStruct((NC, NOUT, CP), jnp.float32),
    mesh=_sc_mesh,
    scratch_types=[
        pltpu.VMEM((CHUNKS, K), jnp.int32),
        pltpu.VMEM((CHUNKS, K), jnp.int32),
        pltpu.VMEM((NBUF, K, CP), jnp.float32),
        pltpu.VMEM((K, CP), jnp.float32),
        pltpu.VMEM_SHARED((NOUT, CP), jnp.float32),
        pltpu.SemaphoreType.DMA,
    ] + [pltpu.SemaphoreType.DMA] * NBUF,
    compiler_params=pltpu.CompilerParams(use_tc_tiling_on_sc=False),
)
def _sc_scatter(p_hbm, src_hbm, dst_hbm, out_hbm, sall, dall, rows, zbuf, acc,
                gsem, *ssems):
    c = lax.axis_index("c")
    s = lax.axis_index("s")
    wid = s * NC + c

    zero16 = jnp.zeros((16,), jnp.float32)

    def _zrow(i, carry):
        zbuf[i, pl.ds(0, 16)] = zero16
        zbuf[i, pl.ds(16, 16)] = zero16
        zbuf[i, pl.ds(32, 16)] = zero16
        return carry

    lax.fori_loop(0, K, _zrow, 0)
    zb = s * RPT
    for q in range(RPT // K):
        pltpu.sync_copy(zbuf, acc.at[pl.ds(zb + q * K, K)])
    _rem = RPT - (RPT // K) * K
    pltpu.sync_copy(zbuf.at[pl.ds(0, _rem)], acc.at[pl.ds(zb + (RPT // K) * K, _rem)])
    plsc.subcore_barrier()

    pltpu.sync_copy(src_hbm.at[wid], sall)
    pltpu.sync_copy(dst_hbm.at[wid], dall)

    def _wait_gather(buf):
        pltpu.make_async_copy(p_hbm.at[pl.ds(0, K)], buf, gsem).wait()

    def _wait_scatter(buf, sem):
        pltpu.make_async_copy(buf, acc.at[pl.ds(0, K)], sem).wait()

    for b in range(NBUF):
        pltpu.async_copy(p_hbm.at[sall.at[b]], rows.at[b], gsem)

    def _grp(gi, carry):
        for b in range(NBUF):
            j = gi * NBUF + b
            buf = rows.at[b]
            _wait_gather(buf)
            pltpu.async_copy(buf, acc.at[dall.at[j]], ssems[b], add=True)

            @pl.when(j + NBUF < CHUNKS)
            def _():
                _wait_scatter(buf, ssems[b])
                pltpu.async_copy(p_hbm.at[sall.at[j + NBUF]], buf, gsem)

        return carry

    lax.fori_loop(0, CHUNKS // NBUF, _grp, 0)
    for b in range(NBUF):
        _wait_scatter(rows.at[b], ssems[b])
    plsc.subcore_barrier()

    pltpu.sync_copy(acc.at[pl.ds(s * RPT, RPT)], out_hbm.at[c, pl.ds(s * RPT, RPT)])


def _enc_body(x_ref, w1_ref, b1_ref, w2_ref, b2_ref, z_ref):
    h = jnp.dot(x_ref[...], w1_ref[...], preferred_element_type=jnp.float32)
    h = jnp.maximum(h + b1_ref[...], 0.0)
    z_ref[...] = jnp.dot(h, w2_ref[...], preferred_element_type=jnp.float32) + b2_ref[...]


_encode = pl.pallas_call(
    _enc_body,
    grid=(GRID,),
    in_specs=[
        pl.BlockSpec((BLK, D_IN), lambda i: (i, 0)),
        pl.BlockSpec((D_IN, HID), lambda i: (0, 0)),
        pl.BlockSpec((1, HID), lambda i: (0, 0)),
        pl.BlockSpec((HID, CP), lambda i: (0, 0)),
        pl.BlockSpec((1, CP), lambda i: (0, 0)),
    ],
    out_specs=pl.BlockSpec((BLK, CP), lambda i: (i, 0)),
    out_shape=jax.ShapeDtypeStruct((N, CP), jnp.float32),
)


def _prep_body(z_ref, o_ref, dinvb_ref, p0_ref):
    deg = o_ref[0][:, 0:1] + o_ref[1][:, 0:1] + 1.0
    dinv = jnp.where(deg > 0, 1.0 / jnp.sqrt(deg), 0.0)
    dinvb = jnp.broadcast_to(dinv, (BLK, CP))
    dinvb_ref[...] = dinvb
    p0_ref[...] = z_ref[...] * dinvb


_prep = pl.pallas_call(
    _prep_body,
    grid=(GRID,),
    in_specs=[
        pl.BlockSpec((BLK, CP), lambda i: (i, 0)),
        pl.BlockSpec((NC, BLK, CP), lambda i: (0, i, 0)),
    ],
    out_specs=[
        pl.BlockSpec((BLK, CP), lambda i: (i, 0)),
        pl.BlockSpec((BLK, CP), lambda i: (i, 0)),
    ],
    out_shape=[
        jax.ShapeDtypeStruct((N, CP), jnp.float32),
        jax.ShapeDtypeStruct((N, CP), jnp.float32),
    ],
)


def _head_body(final, t, o_ref, p_ref, prop_ref, dinvb_ref, act_ref, steps_ref,
               hlp_ref, hv_ref, hent_ref, noise_ref, u_ref,
               pw1_ref, pb1_ref, pw2_ref, pb2_ref, pw3_ref, pb3_ref,
               vw1_ref, vb1_ref, vw2_ref, vb2_ref, vw3_ref, vb3_ref, *outs):
    dinvb = dinvb_ref[...]
    newp = dinvb * (o_ref[0] + o_ref[1] + p_ref[...])
    a = act_ref[...] > 0.0
    prop_new = jnp.where(a, newp, prop_ref[...])

    ph = jnp.maximum(jnp.dot(prop_new, pw1_ref[...], preferred_element_type=jnp.float32) + pb1_ref[...], 0.0)
    ph = jnp.maximum(jnp.dot(ph, pw2_ref[...], preferred_element_type=jnp.float32) + pb2_ref[...], 0.0)
    logit = jnp.dot(ph, pw3_ref[...], preferred_element_type=jnp.float32) + pb3_ref[...]
    p = jax.nn.sigmoid(logit)

    vh = jnp.maximum(jnp.dot(prop_new, vw1_ref[...], preferred_element_type=jnp.float32) + vb1_ref[...], 0.0)
    vh = jnp.maximum(jnp.dot(vh, vw2_ref[...], preferred_element_type=jnp.float32) + vb2_ref[...], 0.0)
    v = jnp.dot(vh, vw3_ref[...], preferred_element_type=jnp.float32) + vb3_ref[...]

    ent = -(p * jnp.log(p + 1e-10) + (1.0 - p) * jnp.log(1.0 - p + 1e-10))
    np_ = jnp.clip(p + noise_ref[...], 0.01, 0.99)
    halt = a & (u_ref[...] < np_)
    hlp_n = jnp.where(halt, jnp.log(np_ + 1e-10), hlp_ref[...])
    hv_n = jnp.where(halt, v, hv_ref[...])
    hent_n = jnp.where(halt, ent, hent_ref[...])
    a_n = a & jnp.logical_not(halt)

    if not final:
        prop_o, pnew_o, act_o, steps_o, hlp_o, hv_o, hent_o = outs
        prop_o[...] = prop_new
        pnew_o[...] = prop_new * dinvb
        act_o[...] = jnp.where(a_n, 1.0, 0.0)
        steps_o[...] = jnp.where(a_n, float(t + 2), steps_ref[...])
        hlp_o[...] = hlp_n
        hv_o[...] = hv_n
        hent_o[...] = hent_n
    else:
        out_o, steps_o, hlp_o, hv_o, hent_o = outs
        steps_o[...] = jnp.where(a_n, float(NITER), steps_ref[...])
        hlp_o[...] = jnp.where(a_n, jnp.log(np_ + 1e-10), hlp_n)
        hv_o[...] = jnp.where(a_n, v, hv_n)
        hent_o[...] = jnp.where(a_n, ent, hent_n)
        x40 = prop_new[:, :C]
        m = jnp.max(x40, axis=1, keepdims=True)
        sh = x40 - lax.stop_gradient(m)
        out_o[...] = sh - jnp.log(jnp.sum(jnp.exp(sh), axis=1, keepdims=True))


def _mk_head(final, t):
    mat = lambda: pl.BlockSpec((BLK, CP), lambda i: (i, 0))
    col = lambda: pl.BlockSpec((BLK, 1), lambda i: (i, 0))
    full = lambda r, c_: pl.BlockSpec((r, c_), lambda i: (0, 0))
    in_specs = [
        pl.BlockSpec((NC, BLK, CP), lambda i: (0, i, 0)),
        mat(), mat(), mat(),
        col(), col(), col(), col(), col(), col(), col(),
        full(CP, 20), full(1, 20), full(20, 10), full(1, 10), full(10, 1), full(1, 1),
        full(CP, 20), full(1, 20), full(20, 10), full(1, 10), full(10, 1), full(1, 1),
    ]
    if not final:
        out_specs = [mat(), mat(), col(), col(), col(), col(), col()]
        out_shape = [jax.ShapeDtypeStruct((N, CP), jnp.float32)] * 2 + \
                    [jax.ShapeDtypeStruct((N, 1), jnp.float32)] * 5
    else:
        out_specs = [pl.BlockSpec((BLK, C), lambda i: (i, 0)),
                     col(), col(), col(), col()]
        out_shape = [jax.ShapeDtypeStruct((N, C), jnp.float32)] + \
                    [jax.ShapeDtypeStruct((N, 1), jnp.float32)] * 4
    return pl.pallas_call(
        functools.partial(_head_body, final, t),
        grid=(GRID,),
        in_specs=in_specs,
        out_specs=out_specs,
        out_shape=out_shape,
    )


_head_step = [_mk_head(False, t) for t in range(NITER - 1)]
_head_final = _mk_head(True, NITER - 1)


def kernel(x, edge_index, W1, b1, W2, b2, pW1, pb1, pW2, pb2, pW3, pb3,
           vW1, vb1, vW2, vb2, vW3, vb3):
    f32 = jnp.float32
    src = edge_index[0].astype(jnp.int32)
    dst = edge_index[1].astype(jnp.int32)
    src_p = jnp.concatenate([src, jnp.full((EPAD - E,), N, jnp.int32)]).reshape(NW, CHUNKS, K)
    dst_p = jnp.concatenate([dst, jnp.zeros((EPAD - E,), jnp.int32)]).reshape(NW, CHUNKS, K)

    W2p = jnp.pad(W2, ((0, 0), (0, CP - C)))
    b2p = jnp.pad(b2, (0, CP - C)).reshape(1, CP)
    pW1p = jnp.pad(pW1, ((0, CP - C), (0, 0)))
    vW1p = jnp.pad(vW1, ((0, CP - C), (0, 0)))
    wts = (pW1p, pb1.reshape(1, -1), pW2, pb2.reshape(1, -1), pW3, pb3.reshape(1, -1),
           vW1p, vb1.reshape(1, -1), vW2, vb2.reshape(1, -1), vW3, vb3.reshape(1, -1))

    rkey = jax.random.key(42)
    noises = [(jax.random.normal(jax.random.fold_in(rkey, 2 * t), (N,), f32) * EXPL).reshape(N, 1)
              for t in range(NITER)]
    us = [jax.random.uniform(jax.random.fold_in(rkey, 2 * t + 1), (N,), f32).reshape(N, 1)
          for t in range(NITER)]

    z = _encode(x, W1, b1.reshape(1, HID), W2p, b2p)

    ones_p = jnp.concatenate([jnp.ones((N, CP), f32), jnp.zeros((NSRC - N, CP), f32)])
    odeg = _sc_scatter(ones_p, src_p, dst_p)
    dinvb, P = _prep(z, odeg)

    zrows = jnp.zeros((NSRC - N, CP), f32)
    prop = z
    act = jnp.ones((N, 1), f32)
    steps = jnp.ones((N, 1), f32)
    hlp = jnp.zeros((N, 1), f32)
    hv = jnp.zeros((N, 1), f32)
    hent = jnp.zeros((N, 1), f32)
    for t in range(NITER):
        o = _sc_scatter(jnp.concatenate([P, zrows]), src_p, dst_p)
        args = (o, P, prop, dinvb, act, steps, hlp, hv, hent, noises[t], us[t]) + wts
        if t < NITER - 1:
            prop, P, act, steps, hlp, hv, hent = _head_step[t](*args)
        else:
            out, steps, hlp, hv, hent = _head_final(*args)
    return (out, steps[:, 0], hlp[:, 0], hv[:, 0], hent[:, 0])
